# Initial kernel scaffold; baseline (speedup 1.0000x reference)
import jax
import jax.numpy as jnp
from jax.experimental import pallas as pl


def kernel(x):
    raise NotImplementedError("write your pallas kernel here")


# speedup vs baseline: 2.2688x; 2.2688x over previous
import jax
import jax.numpy as jnp
from jax.experimental import pallas as pl

_RATIO = 0.1


def _mask_column(C):
    num_mask = int(C * _RATIO)
    perm = jax.random.permutation(jax.random.key(42), C)
    mask_idx = perm[:num_mask]
    return jnp.ones((C, 1), jnp.float32).at[mask_idx, :].set(0.0)


def _body(x_ref, m_ref, o_ref):
    o_ref[...] = x_ref[...] * m_ref[...]


def kernel(x):
    B, C, T = x.shape
    mask = _mask_column(C)
    return pl.pallas_call(
        _body,
        grid=(B,),
        in_specs=[
            pl.BlockSpec((1, C, T), lambda b: (b, 0, 0)),
            pl.BlockSpec((C, 1), lambda b: (0, 0)),
        ],
        out_specs=pl.BlockSpec((1, C, T), lambda b: (b, 0, 0)),
        out_shape=jax.ShapeDtypeStruct((B, C, T), x.dtype),
    )(x, mask)
